# Initial kernel scaffold; baseline (speedup 1.0000x reference)
import jax
import jax.numpy as jnp
from jax.experimental import pallas as pl


def kernel(layer, table):
    raise NotImplementedError("write your pallas kernel here")


# speedup vs baseline: 1.5862x; 1.5862x over previous
import functools
import itertools

import jax
import jax.numpy as jnp
from jax import lax
from jax.experimental import pallas as pl
from jax.experimental.pallas import tpu as pltpu
from jax.experimental.pallas import tpu_sc as plsc

_B = 16384
_D = 64
_G = 4
_BG = _B // _G
_DG = _D * _G

_info = plsc.get_sparse_core_info()
_NC = _info.num_cores
_NS = _info.num_subcores
_L = _info.num_lanes
_NW = _NC * _NS
_B_PER_W = _B // _NW
_BG_PER_W = _BG // _NW

_mesh = plsc.VectorSubcoreMesh(core_axis_name="c", subcore_axis_name="s")


@functools.partial(
    pl.kernel,
    mesh=_mesh,
    out_type=jax.ShapeDtypeStruct((_BG, _DG), jnp.float32),
    scratch_types=[
        pltpu.VMEM((_B_PER_W,), jnp.int32),
        pltpu.VMEM((_BG_PER_W,), jnp.int32),
        pltpu.VMEM((_BG_PER_W, _DG), jnp.float32),
        pltpu.SemaphoreType.DMA,
    ],
)
def _embed_lookup(idx_hbm, ctable_hbm, out_hbm, idx_v, cidx_v, rows_v, sem):
    wid = lax.axis_index("s") * _NC + lax.axis_index("c")
    pltpu.sync_copy(idx_hbm.at[pl.ds(wid * _B_PER_W, _B_PER_W)], idx_v)
    lanes = lax.iota(jnp.int32, _L)
    base = (lanes & (_G - 1)) * _G
    b0 = (lanes >> 2) & 1
    b1 = (lanes >> 3) & 1
    quarter = [(1 - b1) * (1 - b0), (1 - b1) * b0, b1 * (1 - b0), b1 * b0]
    for j in range(_BG_PER_W // _L):
        comb = jnp.zeros((_L,), jnp.int32)
        for k in range(_G):
            a = idx_v[pl.ds(j * _L * _G + k * _L, _L)]
            t = (
                a.at[base].get(mode="promise_in_bounds") * 8
                + a.at[base + 1].get(mode="promise_in_bounds") * 4
                + a.at[base + 2].get(mode="promise_in_bounds") * 2
                + a.at[base + 3].get(mode="promise_in_bounds")
            )
            comb = comb + t * quarter[k]
        cidx_v[pl.ds(j * _L, _L)] = comb
    pltpu.async_copy(ctable_hbm.at[cidx_v], rows_v, sem).wait()
    pltpu.sync_copy(rows_v, out_hbm.at[pl.ds(wid * _BG_PER_W, _BG_PER_W)])


def kernel(layer, table):
    ctable = jnp.stack(
        [
            jnp.concatenate([table[a], table[b], table[c], table[d]])
            for a, b, c, d in itertools.product((0, 1), repeat=_G)
        ]
    )
    out = _embed_lookup(layer, ctable)
    return out.reshape(_B, _D)


# speedup vs baseline: 3.0290x; 1.9097x over previous
import functools

import jax
import jax.numpy as jnp
from jax import lax
from jax.experimental import pallas as pl
from jax.experimental.pallas import tpu as pltpu
from jax.experimental.pallas import tpu_sc as plsc

_B = 16384
_D = 64

_info = plsc.get_sparse_core_info()
_NC = _info.num_cores
_NS = _info.num_subcores
_L = _info.num_lanes
_NW = _NC * _NS
_FG = 8
_NFG = _D // _FG
_NBG = _NW // _NFG
_B_PER_W = _B // _NBG

_mesh = plsc.VectorSubcoreMesh(core_axis_name="c", subcore_axis_name="s")


@functools.partial(
    pl.kernel,
    mesh=_mesh,
    out_type=jax.ShapeDtypeStruct((_D, _B), jnp.float32),
    scratch_types=[
        pltpu.VMEM((_B_PER_W,), jnp.int32),
        pltpu.VMEM((2, _D), jnp.float32),
        pltpu.VMEM((_FG, _B_PER_W), jnp.float32),
        pltpu.SemaphoreType.DMA,
    ],
)
def _embed_lookup_t(idx_hbm, table_hbm, out_hbm, idx_v, table_v, buf_v, sem):
    wid = lax.axis_index("s") * _NC + lax.axis_index("c")
    fg = wid >> 2
    bg = wid & 3
    pltpu.sync_copy(idx_hbm.at[pl.ds(bg * _B_PER_W, _B_PER_W)], idx_v)
    pltpu.sync_copy(table_hbm, table_v)

    lanes = lax.iota(jnp.int32, _L)
    zeros = lanes * 0
    chunk = (fg >> 1) * _L
    t0c = table_v[0, pl.ds(chunk, _L)]
    t1c = table_v[1, pl.ds(chunk, _L)]
    t0b, dtb = [], []
    for k in range(_FG):
        sel = zeros + ((fg * _FG + k) & (_L - 1))
        t0 = t0c.at[sel].get(mode="promise_in_bounds")
        t1 = t1c.at[sel].get(mode="promise_in_bounds")
        t0b.append(t0)
        dtb.append(t1 - t0)

    def batch_body(j, carry):
        lf = idx_v[pl.ds(j * _L, _L)].astype(jnp.float32)
        for k in range(_FG):
            buf_v[k, pl.ds(j * _L, _L)] = t0b[k] + lf * dtb[k]
        return carry

    lax.fori_loop(0, _B_PER_W // _L, batch_body, 0)
    pltpu.sync_copy(
        buf_v,
        out_hbm.at[pl.ds(fg * _FG, _FG), pl.ds(bg * _B_PER_W, _B_PER_W)],
    )


def kernel(layer, table):
    return _embed_lookup_t(layer, table).T


# speedup vs baseline: 3.0992x; 1.0232x over previous
import functools

import jax
import jax.numpy as jnp
from jax import lax
from jax.experimental import pallas as pl
from jax.experimental.pallas import tpu as pltpu
from jax.experimental.pallas import tpu_sc as plsc

_B = 16384
_D = 64

_info = plsc.get_sparse_core_info()
_NC = _info.num_cores
_NS = _info.num_subcores
_L = _info.num_lanes
_NW = _NC * _NS
_FG = 8
_NFG = _D // _FG
_NBG = _NW // _NFG
_B_PER_W = _B // _NBG
_HALF = _B_PER_W // 2

_mesh = plsc.VectorSubcoreMesh(core_axis_name="c", subcore_axis_name="s")


@functools.partial(
    pl.kernel,
    mesh=_mesh,
    out_type=jax.ShapeDtypeStruct((_D, _B), jnp.float32),
    scratch_types=[
        pltpu.VMEM((_B_PER_W,), jnp.int32),
        pltpu.VMEM((2, _D), jnp.float32),
        pltpu.VMEM((_FG, _HALF), jnp.float32),
        pltpu.VMEM((_FG, _HALF), jnp.float32),
        pltpu.SemaphoreType.DMA,
        pltpu.SemaphoreType.DMA,
    ],
)
def _embed_lookup_t(idx_hbm, table_hbm, out_hbm, idx_v, table_v, buf0, buf1, s0, s1):
    wid = lax.axis_index("s") * _NC + lax.axis_index("c")
    fg = wid >> 2
    bg = wid & 3
    pltpu.sync_copy(idx_hbm.at[pl.ds(bg * _B_PER_W, _B_PER_W)], idx_v)
    pltpu.sync_copy(table_hbm, table_v)

    lanes = lax.iota(jnp.int32, _L)
    zeros = lanes * 0
    chunk = (fg >> 1) * _L
    t0c = table_v[0, pl.ds(chunk, _L)]
    t1c = table_v[1, pl.ds(chunk, _L)]
    t0b, dtb = [], []
    for k in range(_FG):
        sel = zeros + ((fg * _FG + k) & (_L - 1))
        t0 = t0c.at[sel].get(mode="promise_in_bounds")
        t1 = t1c.at[sel].get(mode="promise_in_bounds")
        t0b.append(t0)
        dtb.append(t1 - t0)

    def make_body(buf, idx_base):
        def body(j, carry):
            for u in range(2):
                s = pl.ds(j * 2 * _L + u * _L, _L)
                lf = idx_v[pl.ds(idx_base + j * 2 * _L + u * _L, _L)].astype(
                    jnp.float32
                )
                for k in range(_FG):
                    buf[k, s] = t0b[k] + lf * dtb[k]
            return carry
        return body

    row = pl.ds(fg * _FG, _FG)
    lax.fori_loop(0, _HALF // (2 * _L), make_body(buf0, 0), 0)
    cp0 = pltpu.async_copy(
        buf0, out_hbm.at[row, pl.ds(bg * _B_PER_W, _HALF)], s0
    )
    lax.fori_loop(0, _HALF // (2 * _L), make_body(buf1, _HALF), 0)
    cp1 = pltpu.async_copy(
        buf1, out_hbm.at[row, pl.ds(bg * _B_PER_W + _HALF, _HALF)], s1
    )
    cp0.wait()
    cp1.wait()


def kernel(layer, table):
    return _embed_lookup_t(layer, table).T


# speedup vs baseline: 3.1364x; 1.0120x over previous
import functools

import jax
import jax.numpy as jnp
from jax import lax
from jax.experimental import pallas as pl
from jax.experimental.pallas import tpu as pltpu
from jax.experimental.pallas import tpu_sc as plsc

_B = 16384
_D = 64

_info = plsc.get_sparse_core_info()
_NC = _info.num_cores
_NS = _info.num_subcores
_L = _info.num_lanes
_NW = _NC * _NS
_FG = 8
_NFG = _D // _FG
_NBG = _NW // _NFG
_B_PER_W = _B // _NBG
_HALF = _B_PER_W // 2
_UNROLL = 4

_mesh = plsc.VectorSubcoreMesh(core_axis_name="c", subcore_axis_name="s")


@functools.partial(
    pl.kernel,
    mesh=_mesh,
    out_type=jax.ShapeDtypeStruct((_D, _B), jnp.float32),
    scratch_types=[
        pltpu.VMEM((_B_PER_W,), jnp.int32),
        pltpu.VMEM((2, _D), jnp.float32),
        pltpu.VMEM((_FG, _HALF), jnp.float32),
        pltpu.VMEM((_FG, _HALF), jnp.float32),
        pltpu.SemaphoreType.DMA,
        pltpu.SemaphoreType.DMA,
        pltpu.SemaphoreType.DMA,
    ],
)
def _embed_lookup_t(
    idx_hbm, table_hbm, out_hbm, idx_v, table_v, buf0, buf1, s0, s1, si
):
    wid = lax.axis_index("s") * _NC + lax.axis_index("c")
    fg = wid >> 2
    bg = wid & 3
    cpi = pltpu.async_copy(
        idx_hbm.at[pl.ds(bg * _B_PER_W, _B_PER_W)], idx_v, si
    )
    pltpu.sync_copy(table_hbm, table_v)

    lanes = lax.iota(jnp.int32, _L)
    zeros = lanes * 0
    chunk = (fg >> 1) * _L
    t0c = table_v[0, pl.ds(chunk, _L)]
    t1c = table_v[1, pl.ds(chunk, _L)]
    t0b, dtb = [], []
    for k in range(_FG):
        sel = zeros + ((fg * _FG + k) & (_L - 1))
        t0 = t0c.at[sel].get(mode="promise_in_bounds")
        t1 = t1c.at[sel].get(mode="promise_in_bounds")
        t0b.append(t0)
        dtb.append(t1 - t0)
    cpi.wait()

    def make_body(buf, idx_base):
        def body(j, carry):
            for u in range(_UNROLL):
                off = j * _UNROLL * _L + u * _L
                lf = idx_v[pl.ds(idx_base + off, _L)].astype(jnp.float32)
                for k in range(_FG):
                    buf[k, pl.ds(off, _L)] = t0b[k] + lf * dtb[k]
            return carry
        return body

    row = pl.ds(fg * _FG, _FG)
    lax.fori_loop(0, _HALF // (_UNROLL * _L), make_body(buf0, 0), 0)
    cp0 = pltpu.async_copy(
        buf0, out_hbm.at[row, pl.ds(bg * _B_PER_W, _HALF)], s0
    )
    lax.fori_loop(0, _HALF // (_UNROLL * _L), make_body(buf1, _HALF), 0)
    cp1 = pltpu.async_copy(
        buf1, out_hbm.at[row, pl.ds(bg * _B_PER_W + _HALF, _HALF)], s1
    )
    cp0.wait()
    cp1.wait()


def kernel(layer, table):
    return _embed_lookup_t(layer, table).T
